# baseline (speedup 1.0000x reference)
import functools

import jax
import jax.numpy as jnp
from jax import lax
from jax.experimental import pallas as pl
from jax.experimental.pallas import tpu as pltpu
from jax.experimental.pallas import tpu_sc as plsc

N = 10000
NPAD = 10240
E = 160000
EPAD = 163840
ECH = EPAD // 128
CHUNK = 128
NC, NS = 2, 16
TPT = ECH // (NC * NS)
TPT_H = 40
RB = 640
PER_TILE = NPAD // NS
EA = E + N
ECHA = 1344
T0, T1 = 56, 28
TMAX = max(T0, T1)
REG = NS * TMAX
NACC = 10112
PTA = NACC // NS


def _zero_vmem_2d(buf, rows):
    def body(i, _):
        for k in range(128 // 16):
            buf[i, pl.ds(k * 16, 16)] = jnp.zeros((16,), jnp.float32)
        return 0
    lax.fori_loop(0, rows, body, 0)


def _deg_body(dst2d, ew2d, d0, d1, dstv, ewv, degl, rep, zb, acc):
    c = lax.axis_index("c")
    s = lax.axis_index("s")

    def zro(j, _):
        zb[pl.ds(j * 16, 16)] = jnp.zeros((16,), jnp.float32)
        return 0
    lax.fori_loop(0, PER_TILE // 16, zro, 0)
    pltpu.sync_copy(zb, acc.at[pl.ds(s * PER_TILE, PER_TILE)])
    plsc.subcore_barrier()
    base = (c * NS + s) * TPT
    pltpu.sync_copy(dst2d.at[pl.ds(base, TPT)], dstv)
    pltpu.sync_copy(ew2d.at[pl.ds(base, TPT)], ewv)

    def chunk(i, _):
        pltpu.sync_copy(ewv.at[i], acc.at[dstv.at[i]], add=True)
        return 0
    lax.fori_loop(0, TPT, chunk, 0)
    plsc.subcore_barrier()
    pltpu.sync_copy(acc.at[pl.ds(s * PER_TILE, PER_TILE)], degl)

    def repl(t, _):
        v16 = degl[pl.ds(t * 16, 16)]
        for l in range(16):
            v = jnp.broadcast_to(v16[l], (16,))
            for k in range(128 // 16):
                rep[t * 16 + l, pl.ds(k * 16, 16)] = v
        return 0
    lax.fori_loop(0, PER_TILE // 16, repl, 0)

    @pl.when(c == 0)
    def _():
        pltpu.sync_copy(rep, d0.at[pl.ds(s * PER_TILE, PER_TILE)])

    @pl.when(c == 1)
    def _():
        pltpu.sync_copy(rep, d1.at[pl.ds(s * PER_TILE, PER_TILE)])


_deg_kernel = pl.kernel(
    _deg_body,
    out_type=(
        jax.ShapeDtypeStruct((NPAD, 128), jnp.float32),
        jax.ShapeDtypeStruct((NPAD, 128), jnp.float32),
    ),
    mesh=plsc.VectorSubcoreMesh(core_axis_name="c", subcore_axis_name="s", num_cores=NC, num_subcores=NS),
    scratch_types=[
        pltpu.VMEM((TPT, 128), jnp.int32),
        pltpu.VMEM((TPT, 128), jnp.float32),
        pltpu.VMEM((PER_TILE,), jnp.float32),
        pltpu.VMEM((PER_TILE, 128), jnp.float32),
        pltpu.VMEM((PER_TILE,), jnp.float32),
        pltpu.VMEM_SHARED((NPAD,), jnp.float32),
    ],
)


def _agg_body(nr, pk, pw, g, o0, o1,
              ga0, ga1, eb0, eb1, wb0, wb1, acc,
              gsem0, gsem1, ssem0, ssem1, esem0, esem1):
    c = lax.axis_index("c")
    s = lax.axis_index("s")
    tpc = jnp.where(c == 0, T0, T1)
    ebase = jnp.where(c == 0, s * T0, REG + s * T1)
    gbufs = (ga0, ga1)
    ebufs = (eb0, eb1)
    wbufs = (wb0, wb1)
    gsems = (gsem0, gsem1)
    ssems = (ssem0, ssem1)
    esems = (esem0, esem1)

    def start_edges(i, b):
        pltpu.async_copy(pk.at[ebase + i], ebufs[b], esems[b])
        pltpu.async_copy(pw.at[ebase + i], wbufs[b], esems[b])

    def wait_edges(i, b):
        pltpu.make_async_copy(pk.at[ebase + i], ebufs[b], esems[b]).wait()
        pltpu.make_async_copy(pw.at[ebase + i], wbufs[b], esems[b]).wait()

    for r in range(nr):
        gr = g.at[r]

        _zero_vmem_2d(ga0, 128)
        for k in range(4):
            pltpu.sync_copy(ga0, acc.at[pl.ds(s * PTA + k * 128, 128)])
        pltpu.sync_copy(ga0.at[pl.ds(0, 120)],
                        acc.at[pl.ds(s * PTA + 512, 120)])
        plsc.subcore_barrier()

        start_edges(0, 0)
        wait_edges(0, 0)
        pltpu.async_copy(gr.at[eb0.at[0]], ga0, gsem0)

        def halfstep(i, b):
            A, E, W, gs, ss = gbufs[b], ebufs[b], wbufs[b], gsems[b], ssems[b]
            An, En, gn, sn = (gbufs[1 - b], ebufs[1 - b],
                              gsems[1 - b], ssems[1 - b])

            @pl.when(i >= 1)
            def _():
                pltpu.make_async_copy(An, acc.at[En.at[1]], sn).wait()

            @pl.when(i + 1 < tpc)
            def _():
                start_edges(i + 1, 1 - b)
            pltpu.make_async_copy(gr.at[E.at[0]], A, gs).wait()

            def edge(t, _):
                wv = W[pl.ds(t * 16, 16)]
                for l in range(16):
                    e = t * 16 + l
                    w = jnp.broadcast_to(wv[l], (16,))
                    for k in range(128 // 16):
                        A[e, pl.ds(k * 16, 16)] = A[e, pl.ds(k * 16, 16)] * w
                return 0
            lax.fori_loop(0, CHUNK // 16, edge, 0)

            @pl.when(i + 1 < tpc)
            def _():
                wait_edges(i + 1, 1 - b)
                pltpu.async_copy(gr.at[En.at[0]], An, gn)
            pltpu.async_copy(A, acc.at[E.at[1]], ss, add=True)

        def chunk2(o, _):
            halfstep(2 * o, 0)
            halfstep(2 * o + 1, 1)
            return 0
        lax.fori_loop(0, tpc // 2, chunk2, 0)
        pltpu.make_async_copy(ga1, acc.at[eb1.at[1]], ssem1).wait()
        plsc.subcore_barrier()

        @pl.when(c == 0)
        def _():
            pltpu.sync_copy(acc.at[pl.ds(s * PTA, PTA)],
                            o0.at[r].at[pl.ds(s * PTA, PTA)])

        @pl.when(c != 0)
        def _():
            pltpu.sync_copy(acc.at[pl.ds(s * PTA, PTA)],
                            o1.at[r].at[pl.ds(s * PTA, PTA)])


def _make_agg(nr):
    return pl.kernel(
        functools.partial(_agg_body, nr),
        out_type=(
            jax.ShapeDtypeStruct((nr, NPAD, 128), jnp.float32),
            jax.ShapeDtypeStruct((nr, NPAD, 128), jnp.float32),
        ),
        mesh=plsc.VectorSubcoreMesh(core_axis_name="c", subcore_axis_name="s", num_cores=NC, num_subcores=NS),
        scratch_types=[
            pltpu.VMEM((CHUNK, 128), jnp.float32),
            pltpu.VMEM((CHUNK, 128), jnp.float32),
            pltpu.VMEM((2, 128), jnp.int32),
            pltpu.VMEM((2, 128), jnp.int32),
            pltpu.VMEM((128,), jnp.float32),
            pltpu.VMEM((128,), jnp.float32),
            pltpu.VMEM_SHARED((NACC, 128), jnp.float32),
            pltpu.SemaphoreType.DMA,
            pltpu.SemaphoreType.DMA,
            pltpu.SemaphoreType.DMA,
            pltpu.SemaphoreType.DMA,
            pltpu.SemaphoreType.DMA,
            pltpu.SemaphoreType.DMA,
        ],
    )


_agg4 = _make_agg(4)
_agg1 = _make_agg(1)


def _mm1_body(x_ref, w_ref, d0_ref, d1_ref, o_ref):
    dinv = lax.rsqrt(d0_ref[...] + d1_ref[...] + 1.0)
    h = jnp.dot(x_ref[...], w_ref[...], preferred_element_type=jnp.float32)
    o_ref[0] = h * dinv


def _mm1(x, w1, d0, d1):
    return pl.pallas_call(
        _mm1_body,
        grid=(NPAD // RB, 4),
        in_specs=[
            pl.BlockSpec((RB, 1024), lambda i, r: (i, 0)),
            pl.BlockSpec((1024, 128), lambda i, r: (0, r)),
            pl.BlockSpec((RB, 128), lambda i, r: (i, 0)),
            pl.BlockSpec((RB, 128), lambda i, r: (i, 0)),
        ],
        out_specs=pl.BlockSpec((1, RB, 128), lambda i, r: (r, i, 0)),
        out_shape=jax.ShapeDtypeStruct((4, NPAD, 128), jnp.float32),
    )(x, w1, d0, d1)


def _mm2_body(p0_ref, p1_ref, b1_ref, w2_ref, d0_ref, d1_ref, o_ref):
    dinv = lax.rsqrt(d0_ref[...] + d1_ref[...] + 1.0)
    acc = jnp.zeros((RB, 128), jnp.float32)
    for r in range(4):
        a = jax.nn.relu((p0_ref[r] + p1_ref[r]) * dinv + b1_ref[r][None, :])
        acc = acc + jnp.dot(a, w2_ref[r], preferred_element_type=jnp.float32)
    o_ref[...] = acc * dinv


def _mm2(p0, p1, b1r, w2r, d0, d1):
    return pl.pallas_call(
        _mm2_body,
        grid=(NPAD // RB,),
        in_specs=[
            pl.BlockSpec((4, RB, 128), lambda i: (0, i, 0)),
            pl.BlockSpec((4, RB, 128), lambda i: (0, i, 0)),
            pl.BlockSpec((4, 128), lambda i: (0, 0)),
            pl.BlockSpec((4, 128, 128), lambda i: (0, 0, 0)),
            pl.BlockSpec((RB, 128), lambda i: (i, 0)),
            pl.BlockSpec((RB, 128), lambda i: (i, 0)),
        ],
        out_specs=pl.BlockSpec((RB, 128), lambda i: (i, 0)),
        out_shape=jax.ShapeDtypeStruct((NPAD, 128), jnp.float32),
    )(p0, p1, b1r, w2r, d0, d1)


def _head_body(p0_ref, p1_ref, b2_ref, wf_ref, bf_ref, d0_ref, d1_ref, o_ref):
    dinv = lax.rsqrt(d0_ref[...] + d1_ref[...] + 1.0)
    a = jax.nn.relu((p0_ref[...] + p1_ref[...]) * dinv + b2_ref[...])
    logits = jnp.dot(a, wf_ref[...], preferred_element_type=jnp.float32)
    o_ref[...] = jax.nn.sigmoid(logits + bf_ref[...])


def _head(p0, p1, b2m, wfp, bfm, d0, d1):
    return pl.pallas_call(
        _head_body,
        grid=(NPAD // RB,),
        in_specs=[
            pl.BlockSpec((RB, 128), lambda i: (i, 0)),
            pl.BlockSpec((RB, 128), lambda i: (i, 0)),
            pl.BlockSpec((1, 128), lambda i: (0, 0)),
            pl.BlockSpec((128, 128), lambda i: (0, 0)),
            pl.BlockSpec((1, 128), lambda i: (0, 0)),
            pl.BlockSpec((RB, 128), lambda i: (i, 0)),
            pl.BlockSpec((RB, 128), lambda i: (i, 0)),
        ],
        out_specs=pl.BlockSpec((RB, 128), lambda i: (i, 0)),
        out_shape=jax.ShapeDtypeStruct((NPAD, 128), jnp.float32),
    )(p0, p1, b2m, wfp, bfm, d0, d1)


def kernel(bert_output, edge_index, edge_weight, W1, b1, W2, b2, Wfc, bfc):
    src = jnp.pad(edge_index[0], (0, EPAD - E)).reshape(ECH, 128)
    dst = jnp.pad(edge_index[1], (0, EPAD - E)).reshape(ECH, 128)
    ew = jnp.pad(edge_weight, (0, EPAD - E)).reshape(ECH, 128)

    loop = jnp.arange(N, dtype=jnp.int32)
    srca = jnp.pad(jnp.concatenate([edge_index[0], loop]),
                   (0, ECHA * 128 - EA)).reshape(ECHA, 128)
    dsta = jnp.pad(jnp.concatenate([edge_index[1], loop]),
                   (0, ECHA * 128 - EA)).reshape(ECHA, 128)
    ewa = jnp.pad(jnp.concatenate([edge_weight, jnp.ones((N,), jnp.float32)]),
                  (0, ECHA * 128 - EA)).reshape(ECHA, 128)
    packed = jnp.stack([srca, dsta], axis=1)
    cut = NS * T0
    pk = jnp.zeros((2 * REG, 2, 128), jnp.int32)
    pk = lax.dynamic_update_slice(pk, packed[:cut], (0, 0, 0))
    pk = lax.dynamic_update_slice(pk, packed[cut:ECHA], (REG, 0, 0))
    pw = jnp.zeros((2 * REG, 128), jnp.float32)
    pw = lax.dynamic_update_slice(pw, ewa[:cut], (0, 0))
    pw = lax.dynamic_update_slice(pw, ewa[cut:ECHA], (REG, 0))
    x = jnp.pad(bert_output, ((0, NPAD - N), (0, 0)))
    b1r = b1.reshape(4, 128)
    w2r = W2.reshape(4, 128, 128)
    b2m = b2.reshape(1, 128)
    wfp = jnp.pad(Wfc, ((0, 0), (0, 128 - Wfc.shape[1])))
    bfm = jnp.pad(bfc, (0, 128 - bfc.shape[0])).reshape(1, 128)

    d0, d1 = _deg_kernel(dst, ew)
    g1 = _mm1(x, W1, d0, d1)
    p10, p11 = _agg4(pk, pw, g1)
    g2 = _mm2(p10, p11, b1r, w2r, d0, d1)
    p20, p21 = _agg1(pk, pw, g2.reshape(1, NPAD, 128))
    out = _head(p20[0], p21[0], b2m, wfp, bfm, d0, d1)
    return out[:N, :Wfc.shape[1]]


# speedup vs baseline: 1.0034x; 1.0034x over previous
import functools

import jax
import jax.numpy as jnp
from jax import lax
from jax.experimental import pallas as pl
from jax.experimental.pallas import tpu as pltpu
from jax.experimental.pallas import tpu_sc as plsc

N = 10000
NPAD = 10240
E = 160000
EPAD = 163840
ECH = EPAD // 128
CHUNK = 128
NC, NS = 2, 16
TPT = ECH // (NC * NS)
TPT_H = 40
RB = 640
PER_TILE = NPAD // NS
EA = E + N
ECHA = 1344
T0, T1 = 62, 22
TMAX = max(T0, T1)
REG = NS * TMAX
NACC = 10112
PTA = NACC // NS


def _zero_vmem_2d(buf, rows):
    def body(i, _):
        for k in range(128 // 16):
            buf[i, pl.ds(k * 16, 16)] = jnp.zeros((16,), jnp.float32)
        return 0
    lax.fori_loop(0, rows, body, 0)


def _deg_body(dst2d, ew2d, d0, d1, dstv, ewv, degl, rep, zb, acc):
    c = lax.axis_index("c")
    s = lax.axis_index("s")

    def zro(j, _):
        zb[pl.ds(j * 16, 16)] = jnp.zeros((16,), jnp.float32)
        return 0
    lax.fori_loop(0, PER_TILE // 16, zro, 0)
    pltpu.sync_copy(zb, acc.at[pl.ds(s * PER_TILE, PER_TILE)])
    plsc.subcore_barrier()
    base = (c * NS + s) * TPT
    pltpu.sync_copy(dst2d.at[pl.ds(base, TPT)], dstv)
    pltpu.sync_copy(ew2d.at[pl.ds(base, TPT)], ewv)

    def chunk(i, _):
        pltpu.sync_copy(ewv.at[i], acc.at[dstv.at[i]], add=True)
        return 0
    lax.fori_loop(0, TPT, chunk, 0)
    plsc.subcore_barrier()
    pltpu.sync_copy(acc.at[pl.ds(s * PER_TILE, PER_TILE)], degl)

    def repl(t, _):
        v16 = degl[pl.ds(t * 16, 16)]
        for l in range(16):
            v = jnp.broadcast_to(v16[l], (16,))
            for k in range(128 // 16):
                rep[t * 16 + l, pl.ds(k * 16, 16)] = v
        return 0
    lax.fori_loop(0, PER_TILE // 16, repl, 0)

    @pl.when(c == 0)
    def _():
        pltpu.sync_copy(rep, d0.at[pl.ds(s * PER_TILE, PER_TILE)])

    @pl.when(c == 1)
    def _():
        pltpu.sync_copy(rep, d1.at[pl.ds(s * PER_TILE, PER_TILE)])


_deg_kernel = pl.kernel(
    _deg_body,
    out_type=(
        jax.ShapeDtypeStruct((NPAD, 128), jnp.float32),
        jax.ShapeDtypeStruct((NPAD, 128), jnp.float32),
    ),
    mesh=plsc.VectorSubcoreMesh(core_axis_name="c", subcore_axis_name="s", num_cores=NC, num_subcores=NS),
    scratch_types=[
        pltpu.VMEM((TPT, 128), jnp.int32),
        pltpu.VMEM((TPT, 128), jnp.float32),
        pltpu.VMEM((PER_TILE,), jnp.float32),
        pltpu.VMEM((PER_TILE, 128), jnp.float32),
        pltpu.VMEM((PER_TILE,), jnp.float32),
        pltpu.VMEM_SHARED((NPAD,), jnp.float32),
    ],
)


def _agg_body(nr, pk, pw, g, o0, o1,
              ga0, ga1, eb0, eb1, wb0, wb1, acc,
              gsem0, gsem1, ssem0, ssem1, esem0, esem1):
    c = lax.axis_index("c")
    s = lax.axis_index("s")
    tpc = jnp.where(c == 0, T0, T1)
    ebase = jnp.where(c == 0, s * T0, REG + s * T1)
    gbufs = (ga0, ga1)
    ebufs = (eb0, eb1)
    wbufs = (wb0, wb1)
    gsems = (gsem0, gsem1)
    ssems = (ssem0, ssem1)
    esems = (esem0, esem1)

    def start_edges(i, b):
        pltpu.async_copy(pk.at[ebase + i], ebufs[b], esems[b])
        pltpu.async_copy(pw.at[ebase + i], wbufs[b], esems[b])

    def wait_edges(i, b):
        pltpu.make_async_copy(pk.at[ebase + i], ebufs[b], esems[b]).wait()
        pltpu.make_async_copy(pw.at[ebase + i], wbufs[b], esems[b]).wait()

    for r in range(nr):
        gr = g.at[r]

        _zero_vmem_2d(ga0, 128)
        for k in range(4):
            pltpu.sync_copy(ga0, acc.at[pl.ds(s * PTA + k * 128, 128)])
        pltpu.sync_copy(ga0.at[pl.ds(0, 120)],
                        acc.at[pl.ds(s * PTA + 512, 120)])
        plsc.subcore_barrier()

        start_edges(0, 0)
        wait_edges(0, 0)
        pltpu.async_copy(gr.at[eb0.at[0]], ga0, gsem0)

        def halfstep(i, b):
            A, E, W, gs, ss = gbufs[b], ebufs[b], wbufs[b], gsems[b], ssems[b]
            An, En, gn, sn = (gbufs[1 - b], ebufs[1 - b],
                              gsems[1 - b], ssems[1 - b])

            @pl.when(i >= 1)
            def _():
                pltpu.make_async_copy(An, acc.at[En.at[1]], sn).wait()

            @pl.when(i + 1 < tpc)
            def _():
                start_edges(i + 1, 1 - b)
            pltpu.make_async_copy(gr.at[E.at[0]], A, gs).wait()

            def edge(t, _):
                wv = W[pl.ds(t * 16, 16)]
                for l in range(16):
                    e = t * 16 + l
                    w = jnp.broadcast_to(wv[l], (16,))
                    for k in range(128 // 16):
                        A[e, pl.ds(k * 16, 16)] = A[e, pl.ds(k * 16, 16)] * w
                return 0
            lax.fori_loop(0, CHUNK // 16, edge, 0)

            @pl.when(i + 1 < tpc)
            def _():
                wait_edges(i + 1, 1 - b)
                pltpu.async_copy(gr.at[En.at[0]], An, gn)
            pltpu.async_copy(A, acc.at[E.at[1]], ss, add=True)

        def chunk2(o, _):
            halfstep(2 * o, 0)
            halfstep(2 * o + 1, 1)
            return 0
        lax.fori_loop(0, tpc // 2, chunk2, 0)
        pltpu.make_async_copy(ga1, acc.at[eb1.at[1]], ssem1).wait()
        plsc.subcore_barrier()

        @pl.when(c == 0)
        def _():
            pltpu.sync_copy(acc.at[pl.ds(s * PTA, PTA)],
                            o0.at[r].at[pl.ds(s * PTA, PTA)])

        @pl.when(c != 0)
        def _():
            pltpu.sync_copy(acc.at[pl.ds(s * PTA, PTA)],
                            o1.at[r].at[pl.ds(s * PTA, PTA)])


def _make_agg(nr):
    return pl.kernel(
        functools.partial(_agg_body, nr),
        out_type=(
            jax.ShapeDtypeStruct((nr, NPAD, 128), jnp.float32),
            jax.ShapeDtypeStruct((nr, NPAD, 128), jnp.float32),
        ),
        mesh=plsc.VectorSubcoreMesh(core_axis_name="c", subcore_axis_name="s", num_cores=NC, num_subcores=NS),
        scratch_types=[
            pltpu.VMEM((CHUNK, 128), jnp.float32),
            pltpu.VMEM((CHUNK, 128), jnp.float32),
            pltpu.VMEM((2, 128), jnp.int32),
            pltpu.VMEM((2, 128), jnp.int32),
            pltpu.VMEM((128,), jnp.float32),
            pltpu.VMEM((128,), jnp.float32),
            pltpu.VMEM_SHARED((NACC, 128), jnp.float32),
            pltpu.SemaphoreType.DMA,
            pltpu.SemaphoreType.DMA,
            pltpu.SemaphoreType.DMA,
            pltpu.SemaphoreType.DMA,
            pltpu.SemaphoreType.DMA,
            pltpu.SemaphoreType.DMA,
        ],
    )


_agg4 = _make_agg(4)
_agg1 = _make_agg(1)


def _mm1_body(x_ref, w_ref, d0_ref, d1_ref, o_ref):
    dinv = lax.rsqrt(d0_ref[...] + d1_ref[...] + 1.0)
    h = jnp.dot(x_ref[...], w_ref[...], preferred_element_type=jnp.float32)
    o_ref[0] = h * dinv


def _mm1(x, w1, d0, d1):
    return pl.pallas_call(
        _mm1_body,
        grid=(NPAD // RB, 4),
        in_specs=[
            pl.BlockSpec((RB, 1024), lambda i, r: (i, 0)),
            pl.BlockSpec((1024, 128), lambda i, r: (0, r)),
            pl.BlockSpec((RB, 128), lambda i, r: (i, 0)),
            pl.BlockSpec((RB, 128), lambda i, r: (i, 0)),
        ],
        out_specs=pl.BlockSpec((1, RB, 128), lambda i, r: (r, i, 0)),
        out_shape=jax.ShapeDtypeStruct((4, NPAD, 128), jnp.float32),
    )(x, w1, d0, d1)


def _mm2_body(p0_ref, p1_ref, b1_ref, w2_ref, d0_ref, d1_ref, o_ref):
    dinv = lax.rsqrt(d0_ref[...] + d1_ref[...] + 1.0)
    acc = jnp.zeros((RB, 128), jnp.float32)
    for r in range(4):
        a = jax.nn.relu((p0_ref[r] + p1_ref[r]) * dinv + b1_ref[r][None, :])
        acc = acc + jnp.dot(a, w2_ref[r], preferred_element_type=jnp.float32)
    o_ref[...] = acc * dinv


def _mm2(p0, p1, b1r, w2r, d0, d1):
    return pl.pallas_call(
        _mm2_body,
        grid=(NPAD // RB,),
        in_specs=[
            pl.BlockSpec((4, RB, 128), lambda i: (0, i, 0)),
            pl.BlockSpec((4, RB, 128), lambda i: (0, i, 0)),
            pl.BlockSpec((4, 128), lambda i: (0, 0)),
            pl.BlockSpec((4, 128, 128), lambda i: (0, 0, 0)),
            pl.BlockSpec((RB, 128), lambda i: (i, 0)),
            pl.BlockSpec((RB, 128), lambda i: (i, 0)),
        ],
        out_specs=pl.BlockSpec((RB, 128), lambda i: (i, 0)),
        out_shape=jax.ShapeDtypeStruct((NPAD, 128), jnp.float32),
    )(p0, p1, b1r, w2r, d0, d1)


def _head_body(p0_ref, p1_ref, b2_ref, wf_ref, bf_ref, d0_ref, d1_ref, o_ref):
    dinv = lax.rsqrt(d0_ref[...] + d1_ref[...] + 1.0)
    a = jax.nn.relu((p0_ref[...] + p1_ref[...]) * dinv + b2_ref[...])
    logits = jnp.dot(a, wf_ref[...], preferred_element_type=jnp.float32)
    o_ref[...] = jax.nn.sigmoid(logits + bf_ref[...])


def _head(p0, p1, b2m, wfp, bfm, d0, d1):
    return pl.pallas_call(
        _head_body,
        grid=(NPAD // RB,),
        in_specs=[
            pl.BlockSpec((RB, 128), lambda i: (i, 0)),
            pl.BlockSpec((RB, 128), lambda i: (i, 0)),
            pl.BlockSpec((1, 128), lambda i: (0, 0)),
            pl.BlockSpec((128, 128), lambda i: (0, 0)),
            pl.BlockSpec((1, 128), lambda i: (0, 0)),
            pl.BlockSpec((RB, 128), lambda i: (i, 0)),
            pl.BlockSpec((RB, 128), lambda i: (i, 0)),
        ],
        out_specs=pl.BlockSpec((RB, 128), lambda i: (i, 0)),
        out_shape=jax.ShapeDtypeStruct((NPAD, 128), jnp.float32),
    )(p0, p1, b2m, wfp, bfm, d0, d1)


def kernel(bert_output, edge_index, edge_weight, W1, b1, W2, b2, Wfc, bfc):
    src = jnp.pad(edge_index[0], (0, EPAD - E)).reshape(ECH, 128)
    dst = jnp.pad(edge_index[1], (0, EPAD - E)).reshape(ECH, 128)
    ew = jnp.pad(edge_weight, (0, EPAD - E)).reshape(ECH, 128)

    loop = jnp.arange(N, dtype=jnp.int32)
    srca = jnp.pad(jnp.concatenate([edge_index[0], loop]),
                   (0, ECHA * 128 - EA)).reshape(ECHA, 128)
    dsta = jnp.pad(jnp.concatenate([edge_index[1], loop]),
                   (0, ECHA * 128 - EA)).reshape(ECHA, 128)
    ewa = jnp.pad(jnp.concatenate([edge_weight, jnp.ones((N,), jnp.float32)]),
                  (0, ECHA * 128 - EA)).reshape(ECHA, 128)
    packed = jnp.stack([srca, dsta], axis=1)
    cut = NS * T0
    pk = jnp.zeros((2 * REG, 2, 128), jnp.int32)
    pk = lax.dynamic_update_slice(pk, packed[:cut], (0, 0, 0))
    pk = lax.dynamic_update_slice(pk, packed[cut:ECHA], (REG, 0, 0))
    pw = jnp.zeros((2 * REG, 128), jnp.float32)
    pw = lax.dynamic_update_slice(pw, ewa[:cut], (0, 0))
    pw = lax.dynamic_update_slice(pw, ewa[cut:ECHA], (REG, 0))
    x = jnp.pad(bert_output, ((0, NPAD - N), (0, 0)))
    b1r = b1.reshape(4, 128)
    w2r = W2.reshape(4, 128, 128)
    b2m = b2.reshape(1, 128)
    wfp = jnp.pad(Wfc, ((0, 0), (0, 128 - Wfc.shape[1])))
    bfm = jnp.pad(bfc, (0, 128 - bfc.shape[0])).reshape(1, 128)

    d0, d1 = _deg_kernel(dst, ew)
    g1 = _mm1(x, W1, d0, d1)
    p10, p11 = _agg4(pk, pw, g1)
    g2 = _mm2(p10, p11, b1r, w2r, d0, d1)
    p20, p21 = _agg1(pk, pw, g2.reshape(1, NPAD, 128))
    out = _head(p20[0], p21[0], b2m, wfp, bfm, d0, d1)
    return out[:N, :Wfc.shape[1]]


# speedup vs baseline: 1.0035x; 1.0001x over previous
import functools

import jax
import jax.numpy as jnp
from jax import lax
from jax.experimental import pallas as pl
from jax.experimental.pallas import tpu as pltpu
from jax.experimental.pallas import tpu_sc as plsc

N = 10000
NPAD = 10240
E = 160000
EPAD = 163840
ECH = EPAD // 128
CHUNK = 128
NC, NS = 2, 16
TPT = ECH // (NC * NS)
RB = 640
PER_TILE = NPAD // NS
EA = E + N
ECHA = 1344
T0, T1 = 62, 22
TMAX = max(T0, T1)
REG = NS * TMAX
NACC = 10112
PTA = NACC // NS


def _zero_vmem_2d(buf, rows):
    def body(i, _):
        for k in range(128 // 16):
            buf[i, pl.ds(k * 16, 16)] = jnp.zeros((16,), jnp.float32)
        return 0
    lax.fori_loop(0, rows, body, 0)


def _deg_body(dst2d, ew2d, d0, d1, dstv, ewv, degl, rep, zb, acc):
    c = lax.axis_index("c")
    s = lax.axis_index("s")

    def zro(j, _):
        zb[pl.ds(j * 16, 16)] = jnp.zeros((16,), jnp.float32)
        return 0
    lax.fori_loop(0, PER_TILE // 16, zro, 0)
    pltpu.sync_copy(zb, acc.at[pl.ds(s * PER_TILE, PER_TILE)])
    plsc.subcore_barrier()
    base = (c * NS + s) * TPT
    pltpu.sync_copy(dst2d.at[pl.ds(base, TPT)], dstv)
    pltpu.sync_copy(ew2d.at[pl.ds(base, TPT)], ewv)

    def chunk(i, _):
        pltpu.sync_copy(ewv.at[i], acc.at[dstv.at[i]], add=True)
        return 0
    lax.fori_loop(0, TPT, chunk, 0)
    plsc.subcore_barrier()
    pltpu.sync_copy(acc.at[pl.ds(s * PER_TILE, PER_TILE)], degl)

    def repl(t, _):
        v16 = degl[pl.ds(t * 16, 16)]
        for l in range(16):
            v = jnp.broadcast_to(v16[l], (16,))
            for k in range(128 // 16):
                rep[t * 16 + l, pl.ds(k * 16, 16)] = v
        return 0
    lax.fori_loop(0, PER_TILE // 16, repl, 0)

    @pl.when(c == 0)
    def _():
        pltpu.sync_copy(rep, d0.at[pl.ds(s * PER_TILE, PER_TILE)])

    @pl.when(c == 1)
    def _():
        pltpu.sync_copy(rep, d1.at[pl.ds(s * PER_TILE, PER_TILE)])


_deg_kernel = pl.kernel(
    _deg_body,
    out_type=(
        jax.ShapeDtypeStruct((NPAD, 128), jnp.float32),
        jax.ShapeDtypeStruct((NPAD, 128), jnp.float32),
    ),
    mesh=plsc.VectorSubcoreMesh(core_axis_name="c", subcore_axis_name="s", num_cores=NC, num_subcores=NS),
    scratch_types=[
        pltpu.VMEM((TPT, 128), jnp.int32),
        pltpu.VMEM((TPT, 128), jnp.float32),
        pltpu.VMEM((PER_TILE,), jnp.float32),
        pltpu.VMEM((PER_TILE, 128), jnp.float32),
        pltpu.VMEM((PER_TILE,), jnp.float32),
        pltpu.VMEM_SHARED((NPAD,), jnp.float32),
    ],
)


def _agg_body(nr, pk, pw, g, o0, o1,
              ga0, ga1, eb0, eb1, wb0, wb1, acc,
              gsem0, gsem1, ssem0, ssem1, esem0, esem1):
    c = lax.axis_index("c")
    s = lax.axis_index("s")
    tpc = jnp.where(c == 0, T0, T1)
    ebase = jnp.where(c == 0, s * T0, REG + s * T1)
    gbufs = (ga0, ga1)
    ebufs = (eb0, eb1)
    wbufs = (wb0, wb1)
    gsems = (gsem0, gsem1)
    ssems = (ssem0, ssem1)
    esems = (esem0, esem1)

    def start_edges(i, b):
        pltpu.async_copy(pk.at[ebase + i], ebufs[b], esems[b])
        pltpu.async_copy(pw.at[ebase + i], wbufs[b], esems[b])

    def wait_edges(i, b):
        pltpu.make_async_copy(pk.at[ebase + i], ebufs[b], esems[b]).wait()
        pltpu.make_async_copy(pw.at[ebase + i], wbufs[b], esems[b]).wait()

    for r in range(nr):
        gr = g.at[r]

        _zero_vmem_2d(ga0, 128)
        for k in range(4):
            pltpu.sync_copy(ga0, acc.at[pl.ds(s * PTA + k * 128, 128)])
        pltpu.sync_copy(ga0.at[pl.ds(0, 120)],
                        acc.at[pl.ds(s * PTA + 512, 120)])
        plsc.subcore_barrier()

        start_edges(0, 0)
        wait_edges(0, 0)
        pltpu.async_copy(gr.at[eb0.at[0]], ga0, gsem0)

        def halfstep(i, b):
            A, E, W, gs, ss = gbufs[b], ebufs[b], wbufs[b], gsems[b], ssems[b]
            An, En, gn, sn = (gbufs[1 - b], ebufs[1 - b],
                              gsems[1 - b], ssems[1 - b])

            @pl.when(i >= 1)
            def _():
                pltpu.make_async_copy(An, acc.at[En.at[1]], sn).wait()

            @pl.when(i + 1 < tpc)
            def _():
                start_edges(i + 1, 1 - b)
            pltpu.make_async_copy(gr.at[E.at[0]], A, gs).wait()

            def edge(t, _):
                wv = W[pl.ds(t * 16, 16)]
                for l in range(16):
                    e = t * 16 + l
                    w = jnp.broadcast_to(wv[l], (16,))
                    for k in range(128 // 16):
                        A[e, pl.ds(k * 16, 16)] = A[e, pl.ds(k * 16, 16)] * w
                return 0
            lax.fori_loop(0, CHUNK // 16, edge, 0)

            @pl.when(i + 1 < tpc)
            def _():
                wait_edges(i + 1, 1 - b)
                pltpu.async_copy(gr.at[En.at[0]], An, gn)
            pltpu.async_copy(A, acc.at[E.at[1]], ss, add=True)

        def chunk2(o, _):
            halfstep(2 * o, 0)
            halfstep(2 * o + 1, 1)
            return 0
        lax.fori_loop(0, tpc // 2, chunk2, 0)
        pltpu.make_async_copy(ga1, acc.at[eb1.at[1]], ssem1).wait()
        plsc.subcore_barrier()

        @pl.when(c == 0)
        def _():
            pltpu.sync_copy(acc.at[pl.ds(s * PTA, PTA)],
                            o0.at[r].at[pl.ds(s * PTA, PTA)])

        @pl.when(c != 0)
        def _():
            pltpu.sync_copy(acc.at[pl.ds(s * PTA, PTA)],
                            o1.at[r].at[pl.ds(s * PTA, PTA)])


def _make_agg(nr):
    return pl.kernel(
        functools.partial(_agg_body, nr),
        out_type=(
            jax.ShapeDtypeStruct((nr, NPAD, 128), jnp.float32),
            jax.ShapeDtypeStruct((nr, NPAD, 128), jnp.float32),
        ),
        mesh=plsc.VectorSubcoreMesh(core_axis_name="c", subcore_axis_name="s", num_cores=NC, num_subcores=NS),
        scratch_types=[
            pltpu.VMEM((CHUNK, 128), jnp.float32),
            pltpu.VMEM((CHUNK, 128), jnp.float32),
            pltpu.VMEM((2, 128), jnp.int32),
            pltpu.VMEM((2, 128), jnp.int32),
            pltpu.VMEM((128,), jnp.float32),
            pltpu.VMEM((128,), jnp.float32),
            pltpu.VMEM_SHARED((NACC, 128), jnp.float32),
            pltpu.SemaphoreType.DMA,
            pltpu.SemaphoreType.DMA,
            pltpu.SemaphoreType.DMA,
            pltpu.SemaphoreType.DMA,
            pltpu.SemaphoreType.DMA,
            pltpu.SemaphoreType.DMA,
        ],
    )


_agg4 = _make_agg(4)
_agg1 = _make_agg(1)


def _mm1_body(x_ref, w_ref, d0_ref, d1_ref, o_ref):
    dinv = lax.rsqrt(d0_ref[...] + d1_ref[...] + 1.0)
    h = jnp.dot(x_ref[...], w_ref[...], preferred_element_type=jnp.float32)
    o_ref[0] = h * dinv


def _mm1(x, w1, d0, d1):
    return pl.pallas_call(
        _mm1_body,
        grid=(NPAD // RB, 4),
        in_specs=[
            pl.BlockSpec((RB, 1024), lambda i, r: (i, 0)),
            pl.BlockSpec((1024, 128), lambda i, r: (0, r)),
            pl.BlockSpec((RB, 128), lambda i, r: (i, 0)),
            pl.BlockSpec((RB, 128), lambda i, r: (i, 0)),
        ],
        out_specs=pl.BlockSpec((1, RB, 128), lambda i, r: (r, i, 0)),
        out_shape=jax.ShapeDtypeStruct((4, NPAD, 128), jnp.float32),
    )(x, w1, d0, d1)


def _mm2_body(p0_ref, p1_ref, b1_ref, w2_ref, d0_ref, d1_ref, o_ref):
    dinv = lax.rsqrt(d0_ref[...] + d1_ref[...] + 1.0)
    acc = jnp.zeros((RB, 128), jnp.float32)
    for r in range(4):
        a = jax.nn.relu((p0_ref[r] + p1_ref[r]) * dinv + b1_ref[r][None, :])
        acc = acc + jnp.dot(a, w2_ref[r], preferred_element_type=jnp.float32)
    o_ref[...] = acc * dinv


def _mm2(p0, p1, b1r, w2r, d0, d1):
    return pl.pallas_call(
        _mm2_body,
        grid=(NPAD // RB,),
        in_specs=[
            pl.BlockSpec((4, RB, 128), lambda i: (0, i, 0)),
            pl.BlockSpec((4, RB, 128), lambda i: (0, i, 0)),
            pl.BlockSpec((4, 128), lambda i: (0, 0)),
            pl.BlockSpec((4, 128, 128), lambda i: (0, 0, 0)),
            pl.BlockSpec((RB, 128), lambda i: (i, 0)),
            pl.BlockSpec((RB, 128), lambda i: (i, 0)),
        ],
        out_specs=pl.BlockSpec((RB, 128), lambda i: (i, 0)),
        out_shape=jax.ShapeDtypeStruct((NPAD, 128), jnp.float32),
    )(p0, p1, b1r, w2r, d0, d1)


def _head_body(p0_ref, p1_ref, b2_ref, wf_ref, bf_ref, d0_ref, d1_ref, o_ref):
    dinv = lax.rsqrt(d0_ref[...] + d1_ref[...] + 1.0)
    a = jax.nn.relu((p0_ref[...] + p1_ref[...]) * dinv + b2_ref[...])
    logits = jnp.dot(a, wf_ref[...], preferred_element_type=jnp.float32)
    o_ref[...] = jax.nn.sigmoid(logits + bf_ref[...])


def _head(p0, p1, b2m, wfp, bfm, d0, d1):
    return pl.pallas_call(
        _head_body,
        grid=(NPAD // RB,),
        in_specs=[
            pl.BlockSpec((RB, 128), lambda i: (i, 0)),
            pl.BlockSpec((RB, 128), lambda i: (i, 0)),
            pl.BlockSpec((1, 128), lambda i: (0, 0)),
            pl.BlockSpec((128, 128), lambda i: (0, 0)),
            pl.BlockSpec((1, 128), lambda i: (0, 0)),
            pl.BlockSpec((RB, 128), lambda i: (i, 0)),
            pl.BlockSpec((RB, 128), lambda i: (i, 0)),
        ],
        out_specs=pl.BlockSpec((RB, 128), lambda i: (i, 0)),
        out_shape=jax.ShapeDtypeStruct((NPAD, 128), jnp.float32),
    )(p0, p1, b2m, wfp, bfm, d0, d1)


def kernel(bert_output, edge_index, edge_weight, W1, b1, W2, b2, Wfc, bfc):
    dst = jnp.pad(edge_index[1], (0, EPAD - E)).reshape(ECH, 128)
    ew = jnp.pad(edge_weight, (0, EPAD - E)).reshape(ECH, 128)

    loop = jnp.arange(N, dtype=jnp.int32)
    srca = jnp.pad(jnp.concatenate([edge_index[0], loop]),
                   (0, ECHA * 128 - EA)).reshape(ECHA, 128)
    dsta = jnp.pad(jnp.concatenate([edge_index[1], loop]),
                   (0, ECHA * 128 - EA)).reshape(ECHA, 128)
    ewa = jnp.pad(jnp.concatenate([edge_weight, jnp.ones((N,), jnp.float32)]),
                  (0, ECHA * 128 - EA)).reshape(ECHA, 128)
    packed = jnp.stack([srca, dsta], axis=1)
    cut = NS * T0
    pk = jnp.zeros((2 * REG, 2, 128), jnp.int32)
    pk = lax.dynamic_update_slice(pk, packed[:cut], (0, 0, 0))
    pk = lax.dynamic_update_slice(pk, packed[cut:ECHA], (REG, 0, 0))
    pw = jnp.zeros((2 * REG, 128), jnp.float32)
    pw = lax.dynamic_update_slice(pw, ewa[:cut], (0, 0))
    pw = lax.dynamic_update_slice(pw, ewa[cut:ECHA], (REG, 0))
    x = jnp.pad(bert_output, ((0, NPAD - N), (0, 0)))
    b1r = b1.reshape(4, 128)
    w2r = W2.reshape(4, 128, 128)
    b2m = b2.reshape(1, 128)
    wfp = jnp.pad(Wfc, ((0, 0), (0, 128 - Wfc.shape[1])))
    bfm = jnp.pad(bfc, (0, 128 - bfc.shape[0])).reshape(1, 128)

    d0, d1 = _deg_kernel(dst, ew)
    g1 = _mm1(x, W1, d0, d1)
    p10, p11 = _agg4(pk, pw, g1)
    g2 = _mm2(p10, p11, b1r, w2r, d0, d1)
    p20, p21 = _agg1(pk, pw, g2.reshape(1, NPAD, 128))
    out = _head(p20[0], p21[0], b2m, wfp, bfm, d0, d1)
    return out[:N, :Wfc.shape[1]]
